# baseline (speedup 1.0000x reference)
import functools

import jax
import jax.numpy as jnp
from jax.experimental import pallas as pl
from jax.experimental.pallas import tpu as pltpu

_N_DATA = 1000000
_N_CLASSES = 100
_BATCH = 16384

_ROWS_PER_BLOCK = 2048
_N_BLOCKS = _BATCH // _ROWS_PER_BLOCK


def _compute_body(o_ref, t_ref, nt_ref, loss_ref):
    pid = pl.program_id(0)

    x = o_ref[...]
    t = t_ref[...]
    m = jnp.max(x, axis=1, keepdims=True)
    e = jnp.exp(x - m)
    s = jnp.sum(e, axis=1, keepdims=True)
    p = e / s
    logp = (x - m) - jnp.log(s)
    block_loss = jnp.sum(t * logp)

    r = jnp.where(t > 0, p, jnp.zeros_like(p))
    nt = r / jnp.sum(r, axis=1, keepdims=True)
    nt_ref[...] = nt

    @pl.when(pid == 0)
    def _():
        loss_ref[0, 0] = 0.0

    loss_ref[0, 0] += -block_loss / _BATCH


def _compute_tc(output1, target):
    nt, loss = pl.pallas_call(
        _compute_body,
        grid=(_N_BLOCKS,),
        in_specs=[
            pl.BlockSpec((_ROWS_PER_BLOCK, _N_CLASSES), lambda i: (i, 0)),
            pl.BlockSpec((_ROWS_PER_BLOCK, _N_CLASSES), lambda i: (i, 0)),
        ],
        out_specs=[
            pl.BlockSpec((_ROWS_PER_BLOCK, _N_CLASSES), lambda i: (i, 0)),
            pl.BlockSpec(memory_space=pltpu.SMEM, block_shape=(1, 1),
                         index_map=lambda i: (0, 0)),
        ],
        out_shape=[
            jax.ShapeDtypeStruct((_BATCH, _N_CLASSES), jnp.float32),
            jax.ShapeDtypeStruct((1, 1), jnp.float32),
        ],
    )(output1, target)
    return loss[0, 0], nt


def kernel(output1, index, confidence):
    target = jnp.take(confidence, index, axis=0)
    loss, new_target = _compute_tc(output1, target)

    order = jnp.argsort(index, stable=True)
    si = index[order]
    is_last = jnp.concatenate(
        [si[1:] != si[:-1], jnp.ones((1,), dtype=bool)])
    live = jnp.zeros((_BATCH,), dtype=bool).at[order].set(is_last)
    idx2 = jnp.where(live, index, _N_DATA)
    new_confidence = confidence.at[idx2].set(new_target, mode="drop")
    return loss, new_confidence
